# baseline (speedup 1.0000x reference)
import functools

import jax
import jax.numpy as jnp
from jax import lax
from jax.experimental import pallas as pl
from jax.experimental.pallas import tpu as pltpu
from jax.experimental.pallas import tpu_sc as plsc

B = 16384
J = 55
NE = 21
XW = J * 16
EW = NE * 16
OW = XW + EW

NC = 2
NS = 16
NW = NC * NS
ROWS_PER_W = B // NW
CHUNK = 64
NCHUNKS = ROWS_PER_W // CHUNK
GROUPS = CHUNK // 16


def _splat(vec, i):
    return jnp.take_along_axis(vec, jnp.full((16,), i, jnp.int32), axis=0)


def _sc_body(x_hbm, idx_hbm, e_hbm, out_hbm, x_v, ext_v, idx_v, e_v):
    c = lax.axis_index("c")
    s = lax.axis_index("s")
    wid = s * NC + c
    base = wid * ROWS_PER_W

    pltpu.sync_copy(idx_hbm, idx_v)
    pltpu.sync_copy(e_hbm, e_v)
    idx_lo = idx_v[0:16]
    idx_hi = idx_v[16:32]
    iota = lax.iota(jnp.int32, 16)

    def chunk_body(ci, _):
        row0 = base + ci * CHUNK
        pltpu.sync_copy(x_hbm.at[pl.ds(row0, CHUNK)], x_v)
        pltpu.sync_copy(x_v, out_hbm.at[pl.ds(row0, CHUNK), pl.ds(0, XW)])

        def group_body(g, carry):
            rowv = jnp.full((16,), g * 16, jnp.int32) + iota
            for k in range(NE):
                src = idx_lo if k < 16 else idx_hi
                col0 = _splat(src, k % 16) * 16
                evec = e_v[k * 16:(k + 1) * 16]
                gs = [plsc.load_gather(x_v, [rowv, col0 + j]) for j in range(16)]
                for e in range(16):
                    r4 = (e // 4) * 4
                    cc = e % 4
                    acc = gs[r4] * _splat(evec, cc)
                    for cp in range(1, 4):
                        acc = acc + gs[r4 + cp] * _splat(evec, cp * 4 + cc)
                    plsc.store_scatter(
                        ext_v, [rowv, jnp.full((16,), k * 16 + e, jnp.int32)], acc)
            return carry

        lax.fori_loop(0, GROUPS, group_body, 0)
        pltpu.sync_copy(ext_v, out_hbm.at[pl.ds(row0, CHUNK), pl.ds(XW, EW)])
        return _

    lax.fori_loop(0, NCHUNKS, chunk_body, 0)


@jax.jit
def _run(x, idx_pad, e_flat):
    mesh = plsc.VectorSubcoreMesh(
        core_axis_name="c", subcore_axis_name="s", num_cores=NC, num_subcores=NS)
    return pl.kernel(
        _sc_body,
        out_type=jax.ShapeDtypeStruct((B, OW), jnp.float32),
        mesh=mesh,
        scratch_types=[
            pltpu.VMEM((CHUNK, XW), jnp.float32),
            pltpu.VMEM((CHUNK, EW), jnp.float32),
            pltpu.VMEM((32,), jnp.int32),
            pltpu.VMEM((EW,), jnp.float32),
        ],
        compiler_params=pltpu.CompilerParams(
            use_tc_tiling_on_sc=False, needs_layout_passes=False),
    )(x, idx_pad, e_flat)


def kernel(joints_transforms, extra_joint_parent_indices, extra_joint_transforms):
    x = joints_transforms.reshape(B, XW)
    idx = extra_joint_parent_indices.astype(jnp.int32)
    idx_pad = jnp.concatenate([idx, jnp.zeros((32 - NE,), jnp.int32)])
    e_flat = extra_joint_transforms.reshape(EW)
    out = _run(x, idx_pad, e_flat)
    return out.reshape(B, J + NE, 4, 4)
